# baseline (speedup 1.0000x reference)
import functools

import jax
import jax.numpy as jnp
from jax import lax
from jax.experimental import pallas as pl
from jax.experimental.pallas import tpu as pltpu
from jax.experimental.pallas import tpu_sc as plsc

N = 10000
E = 320000
D = 128
EDGE_DIM = 16
NC = 2
NS = 16
K = 40
EDGES_PER_TILE = E // (NC * NS)
GE = 2000
GC = GE // K
NG = EDGES_PER_TILE // GE
NPAD = 10240
ROWS_PER_TILE = NPAD // NS
RSUB = 32
NSUB = ROWS_PER_TILE // RSUB


EB = 6400
EA_STEPS = E // EB


def _mlps_body(ea_in_ref, x_ref, eew1_ref, eeb1_ref, eew2_ref, eeb2_ref,
               new1_ref, neb1_ref, new2_ref, neb2_ref, ea_ref, h_ref):
    i = pl.program_id(0)

    @pl.when(i < EA_STEPS)
    def _():
        t = jnp.dot(ea_in_ref[...], eew1_ref[...],
                    preferred_element_type=jnp.float32)
        t = jnp.maximum(t + eeb1_ref[...], 0.0)
        o = jnp.dot(t, eew2_ref[...], preferred_element_type=jnp.float32)
        ea_ref[...] = o + eeb2_ref[...]

    @pl.when(i == EA_STEPS)
    def _():
        t = jnp.dot(x_ref[...], new1_ref[...],
                    preferred_element_type=jnp.float32)
        t = jnp.maximum(t + neb1_ref[...], 0.0)
        o = jnp.dot(t, new2_ref[...], preferred_element_type=jnp.float32)
        h_ref[...] = o + neb2_ref[...]


def _mlps(edge_attr, x, ee_w1, ee_b1, ee_w2, ee_b2, ne_w1, ne_b1, ne_w2,
          ne_b2):
    full = lambda shape: pl.BlockSpec(shape, lambda i: tuple(0 for _ in shape))
    ea_map = lambda i: (jnp.minimum(i, EA_STEPS - 1), 0)
    return pl.pallas_call(
        _mlps_body,
        grid=(EA_STEPS + 1,),
        in_specs=[
            pl.BlockSpec((EB, EDGE_DIM), ea_map),
            full((N, D)),
            full((EDGE_DIM, D)), full((1, D)), full((D, D)), full((1, D)),
            full((D, D)), full((1, D)), full((D, D)), full((1, D)),
        ],
        out_specs=[
            pl.BlockSpec((EB, D), ea_map),
            full((N, D)),
        ],
        out_shape=[
            jax.ShapeDtypeStruct((E, D), jnp.float32),
            jax.ShapeDtypeStruct((N, D), jnp.float32),
        ],
        compiler_params=pltpu.CompilerParams(
            vmem_limit_bytes=100 * 1024 * 1024),
    )(edge_attr, x, ee_w1, ee_b1.reshape(1, -1), ee_w2, ee_b2.reshape(1, -1),
      ne_w1, ne_b1.reshape(1, -1), ne_w2, ne_b2.reshape(1, -1))


def _sc_aggregate(h, ea, row, col, zrows, zcnt, ones_rows):
    mesh = plsc.VectorSubcoreMesh(core_axis_name="c", subcore_axis_name="s")

    @functools.partial(
        pl.kernel,
        out_type=(
            jax.ShapeDtypeStruct((NC * NPAD, D), jnp.float32),
            jax.ShapeDtypeStruct((NC * NPAD, 16), jnp.float32),
        ),
        mesh=mesh,
        compiler_params=pltpu.CompilerParams(use_tc_tiling_on_sc=False),
        scratch_types=[
            pltpu.VMEM_SHARED((NPAD, D), jnp.float32),
            pltpu.VMEM_SHARED((NPAD, 16), jnp.float32),
            pltpu.VMEM((2, GC, K), jnp.int32),
            pltpu.VMEM((2, GC, K), jnp.int32),
            pltpu.VMEM((2, K, D), jnp.float32),
            pltpu.VMEM((2, K, D), jnp.float32),
            pltpu.VMEM((K, 16), jnp.float32),
            pltpu.VMEM((RSUB, D), jnp.float32),
            pltpu.VMEM((RSUB, 16), jnp.float32),
            pltpu.SemaphoreType.DMA,
            pltpu.SemaphoreType.DMA,
            pltpu.SemaphoreType.DMA,
            pltpu.SemaphoreType.DMA,
            pltpu.SemaphoreType.DMA,
            pltpu.SemaphoreType.DMA,
        ],
    )
    def sc_kernel(h_hbm, ea_hbm, row_hbm, col_hbm, zrows_hbm, zcnt_hbm,
                  ones_hbm, accs_out, cnts_out, acc_sh, cnt_sh, colb, rowb,
                  hbuf, eabuf, onesv, stage, stagec, sem_l0, sem_l1,
                  sem_s0, sem_s1, sem_i, sem_cnt):
        c = lax.axis_index("c")
        s = lax.axis_index("s")
        rbase = s * ROWS_PER_TILE
        pltpu.sync_copy(zrows_hbm, stage)
        pltpu.sync_copy(zcnt_hbm, stagec)
        pltpu.sync_copy(ones_hbm, onesv)
        for j in range(NSUB):
            pltpu.sync_copy(stage, acc_sh.at[pl.ds(rbase + j * RSUB, RSUB)])
            pltpu.sync_copy(stagec, cnt_sh.at[pl.ds(rbase + j * RSUB, RSUB)])
        plsc.subcore_barrier()

        tbase = (c * NS + s) * EDGES_PER_TILE
        slots = ((0, sem_l0, sem_s0), (1, sem_l1, sem_s1))

        def issue_loads(pg, ci, gbase, slot):
            b, sl, _ = slot
            pltpu.async_copy(h_hbm.at[colb.at[pg, ci]], hbuf.at[b], sl)
            pltpu.async_copy(ea_hbm.at[pl.ds(gbase + ci * K, K)],
                             eabuf.at[b], sl)

        def wait_loads(pg, ci, slot):
            b, sl, _ = slot
            pltpu.make_async_copy(h_hbm.at[colb.at[pg, ci]], hbuf.at[b],
                                  sl).wait()
            pltpu.make_async_copy(ea_hbm.at[pl.ds(0, K)], eabuf.at[b],
                                  sl).wait()

        def fire_scatters(pg, ci, slot):
            b, _, ss = slot
            rv = rowb.at[pg, ci]
            pltpu.async_copy(hbuf.at[b], acc_sh.at[rv], ss, add=True)
            pltpu.async_copy(eabuf.at[b], acc_sh.at[rv], ss, add=True)
            pltpu.async_copy(onesv, cnt_sh.at[rv], sem_cnt, add=True)

        def drain_scatters(pg, ci, slot):
            b, _, ss = slot
            rv = rowb.at[pg, ci]
            pltpu.make_async_copy(hbuf.at[b], acc_sh.at[rv], ss).wait()
            pltpu.make_async_copy(eabuf.at[b], acc_sh.at[rv], ss).wait()

        cbase = (c * NS + s) * (NG * GC)
        pltpu.sync_copy(col_hbm.at[pl.ds(cbase, GC)], colb.at[0])
        pltpu.sync_copy(row_hbm.at[pl.ds(cbase, GC)], rowb.at[0])
        for g in range(NG):
            pg = g % 2
            if g + 1 < NG:
                nbase = cbase + (g + 1) * GC
                pltpu.async_copy(col_hbm.at[pl.ds(nbase, GC)],
                                 colb.at[(g + 1) % 2], sem_i)
                pltpu.async_copy(row_hbm.at[pl.ds(nbase, GC)],
                                 rowb.at[(g + 1) % 2], sem_i)
            gbase = tbase + g * GE
            issue_loads(pg, 0, gbase, slots[0])
            issue_loads(pg, 1, gbase, slots[1])

            def body(j, carry):
                for b in (0, 1):
                    ci = 2 * j - 2 + b
                    wait_loads(pg, ci, slots[b])
                    fire_scatters(pg, ci, slots[b])
                    drain_scatters(pg, ci, slots[b])
                    issue_loads(pg, 2 * j + b, gbase, slots[b])
                return carry

            lax.fori_loop(1, GC // 2, body, 0)
            for b in (0, 1):
                ci = GC - 2 + b
                wait_loads(pg, ci, slots[b])
                fire_scatters(pg, ci, slots[b])
                drain_scatters(pg, ci, slots[b])

            def drain_cnt(jj, carry):
                pltpu.make_async_copy(onesv, cnt_sh.at[rowb.at[pg, 0]],
                                      sem_cnt).wait()
                return carry

            lax.fori_loop(0, GC, drain_cnt, 0)
            if g + 1 < NG:
                pltpu.make_async_copy(col_hbm.at[pl.ds(cbase, GC)],
                                      colb.at[(g + 1) % 2], sem_i).wait()
                pltpu.make_async_copy(row_hbm.at[pl.ds(cbase, GC)],
                                      rowb.at[(g + 1) % 2], sem_i).wait()
        plsc.subcore_barrier()
        obase = c * NPAD + rbase
        for j in range(NSUB):
            pltpu.sync_copy(acc_sh.at[pl.ds(rbase + j * RSUB, RSUB)], stage)
            pltpu.sync_copy(stage, accs_out.at[pl.ds(obase + j * RSUB, RSUB)])
            pltpu.sync_copy(cnt_sh.at[pl.ds(rbase + j * RSUB, RSUB)], stagec)
            pltpu.sync_copy(stagec, cnts_out.at[pl.ds(obase + j * RSUB, RSUB)])

    return sc_kernel(h, ea, row, col, zrows, zcnt, ones_rows)


def _finalize_body(accs_ref, cnts_ref, mw_ref, mb_ref, uw_ref, ub_ref, o_ref):
    S = accs_ref[0] + accs_ref[1]
    c16 = cnts_ref[0] + cnts_ref[1]
    cnt = c16[:, 0:1]
    sums = jnp.dot(S, mw_ref[...], preferred_element_type=jnp.float32)
    sums = sums + cnt * mb_ref[...]
    agg = sums / jnp.maximum(cnt, 1.0)
    o = jnp.dot(agg, uw_ref[...], preferred_element_type=jnp.float32)
    o_ref[...] = o + ub_ref[...]


def _finalize(accs, cnts, msg_w, msg_b, upd_w, upd_b, block_rows=1000):
    return pl.pallas_call(
        _finalize_body,
        grid=(N // block_rows,),
        in_specs=[
            pl.BlockSpec((NC, block_rows, D), lambda i: (0, i, 0)),
            pl.BlockSpec((NC, block_rows, 16), lambda i: (0, i, 0)),
            pl.BlockSpec((D, D), lambda i: (0, 0)),
            pl.BlockSpec((1, D), lambda i: (0, 0)),
            pl.BlockSpec((D, D), lambda i: (0, 0)),
            pl.BlockSpec((1, D), lambda i: (0, 0)),
        ],
        out_specs=pl.BlockSpec((block_rows, D), lambda i: (i, 0)),
        out_shape=jax.ShapeDtypeStruct((N, D), jnp.float32),
    )(accs, cnts, msg_w, msg_b.reshape(1, -1), upd_w, upd_b.reshape(1, -1))


def kernel(x, edge_index, edge_attr, ee_w1, ee_b1, ee_w2, ee_b2,
           ne_w1, ne_b1, ne_w2, ne_b2, msg_w, msg_b, upd_w, upd_b):
    ei = edge_index.astype(jnp.int32)
    row = ei[0].reshape(E // K, K)
    col = ei[1].reshape(E // K, K)
    ea, h = _mlps(edge_attr, x, ee_w1, ee_b1, ee_w2, ee_b2,
                  ne_w1, ne_b1, ne_w2, ne_b2)
    zrows = jnp.zeros((RSUB, D), jnp.float32)
    zcnt = jnp.zeros((RSUB, 16), jnp.float32)
    ones_rows = jnp.ones((K, 16), jnp.float32)
    accs, cnts = _sc_aggregate(h, ea, row, col, zrows, zcnt, ones_rows)
    accs = accs.reshape(NC, NPAD, D)
    cnts = cnts.reshape(NC, NPAD, 16)
    out = _finalize(accs, cnts, msg_w, msg_b, upd_w, upd_b)
    return (out, ea)


# speedup vs baseline: 1.0229x; 1.0229x over previous
import functools

import jax
import jax.numpy as jnp
from jax import lax
from jax.experimental import pallas as pl
from jax.experimental.pallas import tpu as pltpu
from jax.experimental.pallas import tpu_sc as plsc

N = 10000
E = 320000
D = 128
EDGE_DIM = 16
NC = 2
NS = 16
K = 40
EDGES_PER_TILE = E // (NC * NS)
GE = 2000
GC = GE // K
NG = EDGES_PER_TILE // GE
NPAD = 10240
ROWS_PER_TILE = NPAD // NS
RSUB = 32
NSUB = ROWS_PER_TILE // RSUB


EB = 12800
EA_STEPS = E // EB


def _mlps_body(ea_in_ref, x_ref, eew1_ref, eeb1_ref, eew2_ref, eeb2_ref,
               new1_ref, neb1_ref, new2_ref, neb2_ref, ea_ref, h_ref):
    i = pl.program_id(0)

    @pl.when(i < EA_STEPS)
    def _():
        t = jnp.dot(ea_in_ref[...], eew1_ref[...],
                    preferred_element_type=jnp.float32)
        t = jnp.maximum(t + eeb1_ref[...], 0.0)
        o = jnp.dot(t, eew2_ref[...], preferred_element_type=jnp.float32)
        ea_ref[...] = o + eeb2_ref[...]

    @pl.when(i == EA_STEPS)
    def _():
        t = jnp.dot(x_ref[...], new1_ref[...],
                    preferred_element_type=jnp.float32)
        t = jnp.maximum(t + neb1_ref[...], 0.0)
        o = jnp.dot(t, new2_ref[...], preferred_element_type=jnp.float32)
        h_ref[...] = o + neb2_ref[...]


def _mlps(edge_attr, x, ee_w1, ee_b1, ee_w2, ee_b2, ne_w1, ne_b1, ne_w2,
          ne_b2):
    full = lambda shape: pl.BlockSpec(shape, lambda i: tuple(0 for _ in shape))
    ea_map = lambda i: (jnp.minimum(i, EA_STEPS - 1), 0)
    return pl.pallas_call(
        _mlps_body,
        grid=(EA_STEPS + 1,),
        in_specs=[
            pl.BlockSpec((EB, EDGE_DIM), ea_map),
            full((N, D)),
            full((EDGE_DIM, D)), full((1, D)), full((D, D)), full((1, D)),
            full((D, D)), full((1, D)), full((D, D)), full((1, D)),
        ],
        out_specs=[
            pl.BlockSpec((EB, D), ea_map),
            full((N, D)),
        ],
        out_shape=[
            jax.ShapeDtypeStruct((E, D), jnp.float32),
            jax.ShapeDtypeStruct((N, D), jnp.float32),
        ],
        compiler_params=pltpu.CompilerParams(
            vmem_limit_bytes=100 * 1024 * 1024),
    )(edge_attr, x, ee_w1, ee_b1.reshape(1, -1), ee_w2, ee_b2.reshape(1, -1),
      ne_w1, ne_b1.reshape(1, -1), ne_w2, ne_b2.reshape(1, -1))


def _sc_aggregate(h, ea, row, col, zrows, zcnt, ones_rows):
    mesh = plsc.VectorSubcoreMesh(core_axis_name="c", subcore_axis_name="s")

    @functools.partial(
        pl.kernel,
        out_type=(
            jax.ShapeDtypeStruct((NC * NPAD, D), jnp.float32),
            jax.ShapeDtypeStruct((NC * NPAD, 16), jnp.float32),
        ),
        mesh=mesh,
        compiler_params=pltpu.CompilerParams(use_tc_tiling_on_sc=False),
        scratch_types=[
            pltpu.VMEM_SHARED((NPAD, D), jnp.float32),
            pltpu.VMEM_SHARED((NPAD, 16), jnp.float32),
            pltpu.VMEM((2, GC, K), jnp.int32),
            pltpu.VMEM((2, GC, K), jnp.int32),
            pltpu.VMEM((2, K, D), jnp.float32),
            pltpu.VMEM((2, K, D), jnp.float32),
            pltpu.VMEM((K, 16), jnp.float32),
            pltpu.VMEM((RSUB, D), jnp.float32),
            pltpu.VMEM((RSUB, 16), jnp.float32),
            pltpu.SemaphoreType.DMA,
            pltpu.SemaphoreType.DMA,
            pltpu.SemaphoreType.DMA,
            pltpu.SemaphoreType.DMA,
            pltpu.SemaphoreType.DMA,
            pltpu.SemaphoreType.DMA,
        ],
    )
    def sc_kernel(h_hbm, ea_hbm, row_hbm, col_hbm, zrows_hbm, zcnt_hbm,
                  ones_hbm, accs_out, cnts_out, acc_sh, cnt_sh, colb, rowb,
                  hbuf, eabuf, onesv, stage, stagec, sem_l0, sem_l1,
                  sem_s0, sem_s1, sem_i, sem_cnt):
        c = lax.axis_index("c")
        s = lax.axis_index("s")
        rbase = s * ROWS_PER_TILE
        pltpu.sync_copy(zrows_hbm, stage)
        pltpu.sync_copy(zcnt_hbm, stagec)
        pltpu.sync_copy(ones_hbm, onesv)
        for j in range(NSUB):
            pltpu.sync_copy(stage, acc_sh.at[pl.ds(rbase + j * RSUB, RSUB)])
            pltpu.sync_copy(stagec, cnt_sh.at[pl.ds(rbase + j * RSUB, RSUB)])
        plsc.subcore_barrier()

        tbase = (c * NS + s) * EDGES_PER_TILE
        slots = ((0, sem_l0, sem_s0), (1, sem_l1, sem_s1))

        def issue_loads(pg, ci, gbase, slot):
            b, sl, _ = slot
            pltpu.async_copy(h_hbm.at[colb.at[pg, ci]], hbuf.at[b], sl)
            pltpu.async_copy(ea_hbm.at[pl.ds(gbase + ci * K, K)],
                             eabuf.at[b], sl)

        def wait_loads(pg, ci, slot):
            b, sl, _ = slot
            pltpu.make_async_copy(h_hbm.at[colb.at[pg, ci]], hbuf.at[b],
                                  sl).wait()
            pltpu.make_async_copy(ea_hbm.at[pl.ds(0, K)], eabuf.at[b],
                                  sl).wait()

        def fire_scatters(pg, ci, slot):
            b, _, ss = slot
            rv = rowb.at[pg, ci]
            pltpu.async_copy(hbuf.at[b], acc_sh.at[rv], ss, add=True)
            pltpu.async_copy(eabuf.at[b], acc_sh.at[rv], ss, add=True)
            pltpu.async_copy(onesv, cnt_sh.at[rv], ss, add=True)

        def drain_scatters(pg, ci, slot):
            b, _, ss = slot
            rv = rowb.at[pg, ci]
            pltpu.make_async_copy(hbuf.at[b], acc_sh.at[rv], ss).wait()
            pltpu.make_async_copy(eabuf.at[b], acc_sh.at[rv], ss).wait()
            pltpu.make_async_copy(onesv, cnt_sh.at[rv], ss).wait()

        cbase = (c * NS + s) * (NG * GC)
        pltpu.sync_copy(col_hbm.at[pl.ds(cbase, GC)], colb.at[0])
        pltpu.sync_copy(row_hbm.at[pl.ds(cbase, GC)], rowb.at[0])
        for g in range(NG):
            pg = g % 2
            if g + 1 < NG:
                nbase = cbase + (g + 1) * GC
                pltpu.async_copy(col_hbm.at[pl.ds(nbase, GC)],
                                 colb.at[(g + 1) % 2], sem_i)
                pltpu.async_copy(row_hbm.at[pl.ds(nbase, GC)],
                                 rowb.at[(g + 1) % 2], sem_i)
            gbase = tbase + g * GE
            issue_loads(pg, 0, gbase, slots[0])
            issue_loads(pg, 1, gbase, slots[1])

            def body(j, carry):
                for b in (0, 1):
                    ci = 2 * j - 2 + b
                    wait_loads(pg, ci, slots[b])
                    fire_scatters(pg, ci, slots[b])
                    drain_scatters(pg, ci, slots[b])
                    issue_loads(pg, 2 * j + b, gbase, slots[b])
                return carry

            lax.fori_loop(1, GC // 2, body, 0)
            for b in (0, 1):
                ci = GC - 2 + b
                wait_loads(pg, ci, slots[b])
                fire_scatters(pg, ci, slots[b])
                drain_scatters(pg, ci, slots[b])
            if g + 1 < NG:
                pltpu.make_async_copy(col_hbm.at[pl.ds(cbase, GC)],
                                      colb.at[(g + 1) % 2], sem_i).wait()
                pltpu.make_async_copy(row_hbm.at[pl.ds(cbase, GC)],
                                      rowb.at[(g + 1) % 2], sem_i).wait()
        plsc.subcore_barrier()
        obase = c * NPAD + rbase
        for j in range(NSUB):
            pltpu.sync_copy(acc_sh.at[pl.ds(rbase + j * RSUB, RSUB)], stage)
            pltpu.sync_copy(stage, accs_out.at[pl.ds(obase + j * RSUB, RSUB)])
            pltpu.sync_copy(cnt_sh.at[pl.ds(rbase + j * RSUB, RSUB)], stagec)
            pltpu.sync_copy(stagec, cnts_out.at[pl.ds(obase + j * RSUB, RSUB)])

    return sc_kernel(h, ea, row, col, zrows, zcnt, ones_rows)


def _finalize_body(accs_ref, cnts_ref, mw_ref, mb_ref, uw_ref, ub_ref, o_ref):
    S = accs_ref[0] + accs_ref[1]
    c16 = cnts_ref[0] + cnts_ref[1]
    cnt = c16[:, 0:1]
    sums = jnp.dot(S, mw_ref[...], preferred_element_type=jnp.float32)
    sums = sums + cnt * mb_ref[...]
    agg = sums / jnp.maximum(cnt, 1.0)
    o = jnp.dot(agg, uw_ref[...], preferred_element_type=jnp.float32)
    o_ref[...] = o + ub_ref[...]


def _finalize(accs, cnts, msg_w, msg_b, upd_w, upd_b, block_rows=1000):
    return pl.pallas_call(
        _finalize_body,
        grid=(N // block_rows,),
        in_specs=[
            pl.BlockSpec((NC, block_rows, D), lambda i: (0, i, 0)),
            pl.BlockSpec((NC, block_rows, 16), lambda i: (0, i, 0)),
            pl.BlockSpec((D, D), lambda i: (0, 0)),
            pl.BlockSpec((1, D), lambda i: (0, 0)),
            pl.BlockSpec((D, D), lambda i: (0, 0)),
            pl.BlockSpec((1, D), lambda i: (0, 0)),
        ],
        out_specs=pl.BlockSpec((block_rows, D), lambda i: (i, 0)),
        out_shape=jax.ShapeDtypeStruct((N, D), jnp.float32),
    )(accs, cnts, msg_w, msg_b.reshape(1, -1), upd_w, upd_b.reshape(1, -1))


def kernel(x, edge_index, edge_attr, ee_w1, ee_b1, ee_w2, ee_b2,
           ne_w1, ne_b1, ne_w2, ne_b2, msg_w, msg_b, upd_w, upd_b):
    ei = edge_index.astype(jnp.int32)
    row = ei[0].reshape(E // K, K)
    col = ei[1].reshape(E // K, K)
    ea, h = _mlps(edge_attr, x, ee_w1, ee_b1, ee_w2, ee_b2,
                  ne_w1, ne_b1, ne_w2, ne_b2)
    zrows = jnp.zeros((RSUB, D), jnp.float32)
    zcnt = jnp.zeros((RSUB, 16), jnp.float32)
    ones_rows = jnp.ones((K, 16), jnp.float32)
    accs, cnts = _sc_aggregate(h, ea, row, col, zrows, zcnt, ones_rows)
    accs = accs.reshape(NC, NPAD, D)
    cnts = cnts.reshape(NC, NPAD, 16)
    out = _finalize(accs, cnts, msg_w, msg_b, upd_w, upd_b)
    return (out, ea)
